# baseline (speedup 1.0000x reference)
import functools

import jax
import jax.numpy as jnp
from jax import lax
from jax.experimental import pallas as pl
from jax.experimental.pallas import tpu as pltpu
from jax.experimental.pallas import tpu_sc as plsc

_GCHUNK = 256
_WIN = 128


def _make_convert(V, D, W, VP, num_cores, num_subcores):
    NW = num_cores * num_subcores
    n_windows = VP // _WIN
    per_w = -(-n_windows // NW)
    mesh = plsc.VectorSubcoreMesh(core_axis_name="c", subcore_axis_name="s")

    @functools.partial(
        pl.kernel,
        mesh=mesh,
        out_type=jax.ShapeDtypeStruct((VP // 8, 8, W), jnp.float32),
        scratch_types=[
            pltpu.VMEM((D, _WIN), jnp.float32),
            pltpu.VMEM((_WIN, W), jnp.float32),
        ],
        compiler_params=pltpu.CompilerParams(needs_layout_passes=False),
    )
    def convert_kernel(tableT_hbm, wide_hbm, slab_v, stage_v):
        wid = lax.axis_index("s") * num_cores + lax.axis_index("c")
        wide_view = wide_hbm.reshape(VP, W)
        col_ids = [
            lax.iota(jnp.int32, 16) + c * 16 for c in range(D // 16)
        ]
        for t in range(per_w):
            w = wid * per_w + t

            @pl.when(w < n_windows)
            def _():
                base = pl.multiple_of(w * _WIN, _WIN)
                pltpu.sync_copy(tableT_hbm.at[:, pl.ds(base, _WIN)], slab_v)

                def row_body(r, carry):
                    rvec = jnp.full((16,), r, dtype=jnp.int32)
                    for c in range(D // 16):
                        stage_v[r, pl.ds(c * 16, 16)] = plsc.load_gather(
                            slab_v, [col_ids[c], rvec]
                        )
                    return carry

                lax.fori_loop(0, _WIN, row_body, 0)
                pltpu.sync_copy(stage_v, wide_view.at[pl.ds(base, _WIN)])

    return convert_kernel


def _make_gather(B, W, VP, num_cores, num_subcores):
    NW = num_cores * num_subcores
    b_per_w = B // NW
    n_chunks = b_per_w // _GCHUNK
    mesh = plsc.VectorSubcoreMesh(core_axis_name="c", subcore_axis_name="s")

    @functools.partial(
        pl.kernel,
        mesh=mesh,
        out_type=jax.ShapeDtypeStruct((B, W), jnp.float32),
        scratch_types=[
            pltpu.VMEM((b_per_w,), jnp.int32),
            pltpu.VMEM((_GCHUNK, W), jnp.float32),
            pltpu.SemaphoreType.DMA,
        ],
    )
    def gather_kernel(idx_hbm, wide_hbm, out_hbm, idx_v, rows_v, sem):
        wid = lax.axis_index("s") * num_cores + lax.axis_index("c")
        base = pl.multiple_of(wid * b_per_w, 8)
        wide_view = wide_hbm.reshape(VP, W)
        pltpu.sync_copy(idx_hbm.at[pl.ds(base, b_per_w)], idx_v)

        def chunk_body(g, carry):
            off = pl.multiple_of(g * _GCHUNK, 8)
            pltpu.async_copy(
                wide_view.at[idx_v.at[pl.ds(off, _GCHUNK)]], rows_v, sem
            ).wait()
            pltpu.sync_copy(rows_v, out_hbm.at[pl.ds(base + off, _GCHUNK)])
            return carry

        lax.fori_loop(0, n_chunks, chunk_body, 0)

    return gather_kernel


def kernel(index, train_score):
    index = index.astype(jnp.int32)
    B = index.shape[0]
    V, D = train_score.shape
    W = 2 * D
    VP = -(-V // _WIN) * _WIN
    info = plsc.get_sparse_core_info()
    convert = _make_convert(V, D, W, VP, info.num_cores, info.num_subcores)
    gather = _make_gather(B, W, VP, info.num_cores, info.num_subcores)
    wide = convert(train_score.T)
    out128 = gather(index, wide)
    return lax.slice(out128, (0, 0), (B, D))


# speedup vs baseline: 2.7869x; 2.7869x over previous
import functools

import jax
import jax.numpy as jnp
from jax import lax
from jax.experimental import pallas as pl
from jax.experimental.pallas import tpu as pltpu
from jax.experimental.pallas import tpu_sc as plsc


def _make_gather(B, V, W, num_cores, num_subcores):
    NW = num_cores * num_subcores
    b_per_w = B // NW
    mesh = plsc.VectorSubcoreMesh(core_axis_name="c", subcore_axis_name="s")

    @functools.partial(
        pl.kernel,
        mesh=mesh,
        out_type=jax.ShapeDtypeStruct((B, W), jnp.float32),
        scratch_types=[
            pltpu.VMEM((b_per_w,), jnp.int32),
            pltpu.VMEM((b_per_w, W), jnp.float32),
            pltpu.SemaphoreType.DMA,
        ],
    )
    def gather_kernel(idx_hbm, wide_hbm, out_hbm, idx_v, rows_v, sem):
        wid = lax.axis_index("s") * num_cores + lax.axis_index("c")
        base = pl.multiple_of(wid * b_per_w, 8)
        pltpu.sync_copy(idx_hbm.at[pl.ds(base, b_per_w)], idx_v)
        pltpu.async_copy(wide_hbm.at[idx_v], rows_v, sem).wait()
        pltpu.sync_copy(rows_v, out_hbm.at[pl.ds(base, b_per_w)])

    return gather_kernel


def kernel(index, train_score):
    index = index.astype(jnp.int32)
    B = index.shape[0]
    V, D = train_score.shape
    W = 2 * D
    wide = jnp.pad(train_score, ((0, 0), (0, W - D)))
    info = plsc.get_sparse_core_info()
    gather = _make_gather(B, V, W, info.num_cores, info.num_subcores)
    out128 = gather(index, wide)
    return lax.slice(out128, (0, 0), (B, D))
